# Initial kernel scaffold; baseline (speedup 1.0000x reference)
import jax
import jax.numpy as jnp
from jax.experimental import pallas as pl


def kernel(x):
    raise NotImplementedError("write your pallas kernel here")


# speedup vs baseline: 11.0428x; 11.0428x over previous
import jax
import jax.numpy as jnp
from jax.experimental import pallas as pl
from jax.experimental.pallas import tpu as pltpu

_K = 512
_N = 32768
_ROWS = 128
_R = 8


def _topk_mask_body(x_ref, o_ref):
    x = x_ref[...]
    u = jax.lax.bitcast_convert_type(x, jnp.uint32)
    neg = (u >> 31) == jnp.uint32(1)
    u = jnp.where(neg, ~u, u | jnp.uint32(0x80000000))

    m = jnp.zeros((_R, 1), jnp.uint32)
    for i in range(31, -1, -1):
        cand = m | jnp.uint32(1 << i)
        cnt = jnp.sum((u >= cand).astype(jnp.int32), axis=1, keepdims=True)
        m = jnp.where(cnt >= _K, cand, m)

    keep = u >= m
    o_ref[...] = jnp.where(keep, jnp.maximum(x, 0.0), 0.0)


@jax.jit
def kernel(x):
    return pl.pallas_call(
        _topk_mask_body,
        grid=(_ROWS // _R,),
        in_specs=[pl.BlockSpec((_R, _N), lambda i: (i, 0))],
        out_specs=pl.BlockSpec((_R, _N), lambda i: (i, 0)),
        out_shape=jax.ShapeDtypeStruct((_ROWS, _N), jnp.float32),
    )(x)


# speedup vs baseline: 20.5864x; 1.8642x over previous
import jax
import jax.numpy as jnp
from jax.experimental import pallas as pl
from jax.experimental.pallas import tpu as pltpu

_K = 512
_N = 32768
_ROWS = 128
_R = 16


def _topk_mask_body(x_ref, o_ref):
    x = x_ref[...]
    u = jax.lax.bitcast_convert_type(x, jnp.uint32)
    neg = (u >> 31) == jnp.uint32(1)
    u = jnp.where(neg, ~u, u | jnp.uint32(0x80000000))

    m = jnp.zeros((_R, 1), jnp.uint32)
    for i in range(31, -1, -1):
        cand = m | jnp.uint32(1 << i)
        cnt = jnp.sum((u >= cand).astype(jnp.int32), axis=1, keepdims=True)
        m = jnp.where(cnt >= _K, cand, m)

    keep = u >= m
    o_ref[...] = jnp.where(keep, jnp.maximum(x, 0.0), 0.0)


@jax.jit
def kernel(x):
    return pl.pallas_call(
        _topk_mask_body,
        grid=(_ROWS // _R,),
        in_specs=[pl.BlockSpec((_R, _N), lambda i: (i, 0))],
        out_specs=pl.BlockSpec((_R, _N), lambda i: (i, 0)),
        out_shape=jax.ShapeDtypeStruct((_ROWS, _N), jnp.float32),
    )(x)


# speedup vs baseline: 23.9943x; 1.1655x over previous
import jax
import jax.numpy as jnp
from jax.experimental import pallas as pl
from jax.experimental.pallas import tpu as pltpu

_K = 512
_N = 32768
_ROWS = 128
_R = 32


def _topk_mask_body(x_ref, o_ref):
    x = x_ref[...]
    u = jax.lax.bitcast_convert_type(x, jnp.uint32)
    neg = (u >> 31) == jnp.uint32(1)
    u = jnp.where(neg, ~u, u | jnp.uint32(0x80000000))

    m = jnp.zeros((_R, 1), jnp.uint32)
    for i in range(31, -1, -1):
        cand = m | jnp.uint32(1 << i)
        cnt = jnp.sum((u >= cand).astype(jnp.int32), axis=1, keepdims=True)
        m = jnp.where(cnt >= _K, cand, m)

    keep = u >= m
    o_ref[...] = jnp.where(keep, jnp.maximum(x, 0.0), 0.0)


@jax.jit
def kernel(x):
    return pl.pallas_call(
        _topk_mask_body,
        grid=(_ROWS // _R,),
        in_specs=[pl.BlockSpec((_R, _N), lambda i: (i, 0))],
        out_specs=pl.BlockSpec((_R, _N), lambda i: (i, 0)),
        out_shape=jax.ShapeDtypeStruct((_ROWS, _N), jnp.float32),
    )(x)
